# baseline (speedup 1.0000x reference)
import functools

import jax
import jax.numpy as jnp
from jax import lax
from jax.experimental import pallas as pl
from jax.experimental.pallas import tpu as pltpu
from jax.experimental.pallas import tpu_sc as plsc

B = 1
D = 128
N = 10000
K = 16

NW = 32
NPAD = 10240
NODES_PER_W = NPAD // NW
CHUNK = 8
NCHUNK = NODES_PER_W // CHUNK
IDX_PER_CHUNK = CHUNK * K
NB = 512
NBLK = NPAD // NB


def _mm_body(x_ref, w_ref, b_ref, yt_ref, zt_ref):
    xb = x_ref[...]
    w1 = w_ref[:, :D]
    wz = w_ref[:, D:] - w1
    dn = (((0,), (1,)), ((), ()))
    yt_ref[...] = lax.dot_general(xb, w1, dn, preferred_element_type=jnp.float32)
    zt_ref[...] = (
        lax.dot_general(xb, wz, dn, preferred_element_type=jnp.float32) + b_ref[...]
    )


def _add_body(x_ref, m_ref, o_ref):
    o_ref[...] = x_ref[...] + m_ref[...].T


NBUF = 4


def _sc_knn(yt_hbm, zt_hbm, idx_hbm, out_hbm, idx_v, gbufs, obufs, zbuf, sems,
            osems):
    wid = lax.axis_index("s") * 2 + lax.axis_index("c")
    base = wid * NODES_PER_W
    pltpu.sync_copy(idx_hbm.at[wid], idx_v)
    pltpu.sync_copy(zt_hbm.at[pl.ds(base, NODES_PER_W)], zbuf)

    def issue(jc, b):
        pltpu.async_copy(yt_hbm.at[idx_v.at[jc]], gbufs[b], sems[b])

    def wait_gather(b):
        pltpu.make_async_copy(
            yt_hbm.at[pl.ds(0, IDX_PER_CHUNK)], gbufs[b], sems[b]).wait()

    def wait_store(b):
        pltpu.make_async_copy(
            zt_hbm.at[pl.ds(0, CHUNK)], obufs[b], osems[b]).wait()

    def compute_store(j, b):
        buf = gbufs[b]
        obuf = obufs[b]

        def node_body(i, _):
            r0 = i * K
            row = j * CHUNK + i
            for g in range(D // 16):
                sl = pl.ds(g * 16, 16)
                m = buf[r0, sl]
                for k in range(1, K):
                    m = jnp.maximum(m, buf[r0 + k, sl])
                obuf[i, sl] = jnp.maximum(m + zbuf[row, sl], 0.0)
            return 0

        lax.fori_loop(0, CHUNK, node_body, 0)
        pltpu.async_copy(
            obufs[b], out_hbm.at[pl.ds(base + j * CHUNK, CHUNK)], osems[b])

    for b in range(NBUF):
        issue(b, b)
        pltpu.async_copy(zt_hbm.at[pl.ds(0, CHUNK)], obufs[b], osems[b])

    def chunk_body(jj, _):
        j0 = jj * NBUF
        for b in range(NBUF):
            wait_gather(b)
            wait_store(b)
            compute_store(j0 + b, b)
            issue(jnp.minimum(j0 + b + NBUF, NCHUNK - 1), b)
        return 0

    lax.fori_loop(0, NCHUNK // NBUF, chunk_body, 0)
    for b in range(NBUF):
        wait_gather(b)
        wait_store(b)


def kernel(x, idx, W, b):
    x2 = x[0]
    x_pad = jnp.pad(x2, ((0, 0), (0, NPAD - N)))
    idx_flat = idx[0].astype(jnp.int32).reshape(-1)
    idx3 = jnp.pad(idx_flat, (0, (NPAD - N) * K)).reshape(NW, NCHUNK, IDX_PER_CHUNK)
    b2 = b.reshape(1, D)

    yt, zt = pl.pallas_call(
        _mm_body,
        grid=(NBLK,),
        in_specs=[
            pl.BlockSpec((D, NB), lambda i: (0, i)),
            pl.BlockSpec((D, 2 * D), lambda i: (0, 0)),
            pl.BlockSpec((1, D), lambda i: (0, 0)),
        ],
        out_specs=[
            pl.BlockSpec((NB, D), lambda i: (i, 0)),
            pl.BlockSpec((NB, D), lambda i: (i, 0)),
        ],
        out_shape=[
            jax.ShapeDtypeStruct((NPAD, D), jnp.float32),
            jax.ShapeDtypeStruct((NPAD, D), jnp.float32),
        ],
    )(x_pad, W, b2)

    mesh = plsc.VectorSubcoreMesh(core_axis_name="c", subcore_axis_name="s")
    mt = pl.kernel(
        _sc_knn,
        out_type=jax.ShapeDtypeStruct((NPAD, D), jnp.float32),
        mesh=mesh,
        scratch_types=[
            pltpu.VMEM((NCHUNK, IDX_PER_CHUNK), jnp.int32),
            [pltpu.VMEM((IDX_PER_CHUNK, D), jnp.float32) for _ in range(NBUF)],
            [pltpu.VMEM((CHUNK, D), jnp.float32) for _ in range(NBUF)],
            pltpu.VMEM((NODES_PER_W, D), jnp.float32),
            [pltpu.SemaphoreType.DMA for _ in range(NBUF)],
            [pltpu.SemaphoreType.DMA for _ in range(NBUF)],
        ],
    )(yt, zt, idx3)

    out = pl.pallas_call(
        _add_body,
        grid=(NBLK,),
        in_specs=[
            pl.BlockSpec((D, NB), lambda i: (0, i)),
            pl.BlockSpec((NB, D), lambda i: (i, 0)),
        ],
        out_specs=pl.BlockSpec((D, NB), lambda i: (0, i)),
        out_shape=jax.ShapeDtypeStruct((D, N), jnp.float32),
    )(x_pad, mt)

    return out[None]


# speedup vs baseline: 2.7528x; 2.7528x over previous
import functools

import jax
import jax.numpy as jnp
from jax import lax
from jax.experimental import pallas as pl
from jax.experimental.pallas import tpu as pltpu
from jax.experimental.pallas import tpu_sc as plsc

B = 1
D = 128
N = 10000
K = 16

NW = 32
NPAD = 10240
NODES_PER_W = NPAD // NW
CHUNK = 8
NCHUNK = NODES_PER_W // CHUNK
IDX_PER_CHUNK = CHUNK * K
NB = 512
NBLK = NPAD // NB
ROWS_PER_TILE = NPAD // 16


def _mm_body(x_ref, w_ref, yt_ref):
    xb = x_ref[...]
    w1 = w_ref[:, :D]
    dn = (((0,), (1,)), ((), ()))
    yt_ref[...] = lax.dot_general(xb, w1, dn, preferred_element_type=jnp.float32)


def _add_body(x_ref, m_ref, w_ref, b_ref, o_ref):
    xb = x_ref[...]
    wz = w_ref[:, D:] - w_ref[:, :D]
    z = lax.dot_general(wz, xb, (((1,), (0,)), ((), ())),
                        preferred_element_type=jnp.float32) + b_ref[...].T
    o_ref[...] = xb + jnp.maximum(z + m_ref[...].T, 0.0)


def _sc_knn(yt_hbm, idx_hbm, out_hbm, idx_v, gbuf_a, gbuf_b, obuf_a, obuf_b,
            yt_sp, sem_a, sem_b):
    cid = lax.axis_index("c")
    sid = lax.axis_index("s")
    wid = sid * 2 + cid
    base = wid * NODES_PER_W
    pltpu.sync_copy(yt_hbm.at[pl.ds(sid * ROWS_PER_TILE, ROWS_PER_TILE)],
                    yt_sp.at[pl.ds(sid * ROWS_PER_TILE, ROWS_PER_TILE)])
    pltpu.sync_copy(idx_hbm.at[wid], idx_v)
    plsc.subcore_barrier()

    def issue(jc, buf, sem):
        pltpu.async_copy(yt_sp.at[idx_v.at[jc]], buf, sem)

    def wait(buf, sem):
        pltpu.make_async_copy(yt_sp.at[pl.ds(0, IDX_PER_CHUNK)], buf, sem).wait()

    def compute_store(j, buf, obuf):
        def node_body(i, _):
            r0 = i * K
            for g in range(D // 16):
                sl = pl.ds(g * 16, 16)
                m = buf[r0, sl]
                for k in range(1, K):
                    m = jnp.maximum(m, buf[r0 + k, sl])
                obuf[i, sl] = m
            return 0

        lax.fori_loop(0, CHUNK, node_body, 0)
        pltpu.sync_copy(obuf, out_hbm.at[pl.ds(base + j * CHUNK, CHUNK)])

    issue(0, gbuf_a, sem_a)
    issue(1, gbuf_b, sem_b)

    def chunk_body(jj, _):
        j0 = jj * 2
        wait(gbuf_a, sem_a)
        compute_store(j0, gbuf_a, obuf_a)
        issue(jnp.minimum(j0 + 2, NCHUNK - 1), gbuf_a, sem_a)
        wait(gbuf_b, sem_b)
        compute_store(j0 + 1, gbuf_b, obuf_b)
        issue(jnp.minimum(j0 + 3, NCHUNK - 1), gbuf_b, sem_b)
        return 0

    lax.fori_loop(0, NCHUNK // 2, chunk_body, 0)
    wait(gbuf_a, sem_a)
    wait(gbuf_b, sem_b)


def kernel(x, idx, W, b):
    x2 = x[0]
    x_pad = jnp.pad(x2, ((0, 0), (0, NPAD - N)))
    idx_flat = idx[0].astype(jnp.int32).reshape(-1)
    idx3 = jnp.pad(idx_flat, (0, (NPAD - N) * K)).reshape(NW, NCHUNK, IDX_PER_CHUNK)
    b2 = b.reshape(1, D)

    yt = pl.pallas_call(
        _mm_body,
        grid=(NBLK,),
        in_specs=[
            pl.BlockSpec((D, NB), lambda i: (0, i)),
            pl.BlockSpec((D, 2 * D), lambda i: (0, 0)),
        ],
        out_specs=pl.BlockSpec((NB, D), lambda i: (i, 0)),
        out_shape=jax.ShapeDtypeStruct((NPAD, D), jnp.float32),
    )(x_pad, W)

    mesh = plsc.VectorSubcoreMesh(core_axis_name="c", subcore_axis_name="s")
    mt = pl.kernel(
        _sc_knn,
        out_type=jax.ShapeDtypeStruct((NPAD, D), jnp.float32),
        mesh=mesh,
        scratch_types=[
            pltpu.VMEM((NCHUNK, IDX_PER_CHUNK), jnp.int32),
            pltpu.VMEM((IDX_PER_CHUNK, D), jnp.float32),
            pltpu.VMEM((IDX_PER_CHUNK, D), jnp.float32),
            pltpu.VMEM((CHUNK, D), jnp.float32),
            pltpu.VMEM((CHUNK, D), jnp.float32),
            pltpu.VMEM_SHARED((NPAD, D), jnp.float32),
            pltpu.SemaphoreType.DMA,
            pltpu.SemaphoreType.DMA,
        ],
    )(yt, idx3)

    out = pl.pallas_call(
        _add_body,
        grid=(NBLK,),
        in_specs=[
            pl.BlockSpec((D, NB), lambda i: (0, i)),
            pl.BlockSpec((NB, D), lambda i: (i, 0)),
            pl.BlockSpec((D, 2 * D), lambda i: (0, 0)),
            pl.BlockSpec((1, D), lambda i: (0, 0)),
        ],
        out_specs=pl.BlockSpec((D, NB), lambda i: (0, i)),
        out_shape=jax.ShapeDtypeStruct((D, N), jnp.float32),
    )(x_pad, mt, W, b2)

    return out[None]


# speedup vs baseline: 2.8100x; 1.0208x over previous
import functools

import jax
import jax.numpy as jnp
from jax import lax
from jax.experimental import pallas as pl
from jax.experimental.pallas import tpu as pltpu
from jax.experimental.pallas import tpu_sc as plsc

B = 1
D = 128
N = 10000
K = 16

NW = 32
NPAD = 10240
NODES_PER_W = NPAD // NW
CHUNK = 8
NCHUNK = NODES_PER_W // CHUNK
IDX_PER_CHUNK = CHUNK * K
NB = 512
NBLK = NPAD // NB
ROWS_PER_TILE = NPAD // 16


def _mm_body(x_ref, w_ref, yt_ref):
    xb = x_ref[...]
    w1 = w_ref[:, :D]
    dn = (((0,), (1,)), ((), ()))
    yt_ref[...] = lax.dot_general(xb, w1, dn, preferred_element_type=jnp.float32)


def _add_body(x_ref, m_ref, w_ref, b_ref, o_ref):
    xb = x_ref[...]
    wz = w_ref[:, D:] - w_ref[:, :D]
    z = lax.dot_general(wz, xb, (((1,), (0,)), ((), ())),
                        preferred_element_type=jnp.float32) + b_ref[...].T
    o_ref[...] = xb + jnp.maximum(z + m_ref[...].T, 0.0)


def _sc_knn(yt_hbm, idx_hbm, out_hbm, idx_v, gbuf_a, gbuf_b, obuf_a, obuf_b,
            yt_sp, sem_a, sem_b):
    cid = lax.axis_index("c")
    sid = lax.axis_index("s")
    wid = sid * 2 + cid
    base = wid * NODES_PER_W
    pltpu.sync_copy(yt_hbm.at[pl.ds(sid * ROWS_PER_TILE, ROWS_PER_TILE)],
                    yt_sp.at[pl.ds(sid * ROWS_PER_TILE, ROWS_PER_TILE)])
    pltpu.sync_copy(idx_hbm.at[wid], idx_v)
    plsc.subcore_barrier()

    def issue(jc, buf, sem):
        pltpu.async_copy(yt_sp.at[idx_v.at[jc]], buf, sem)

    def wait(buf, sem):
        pltpu.make_async_copy(yt_sp.at[pl.ds(0, IDX_PER_CHUNK)], buf, sem).wait()

    def compute_store(j, buf, obuf):
        def node_body(i, _):
            r0 = i * K
            for g in range(D // 16):
                sl = pl.ds(g * 16, 16)
                m = buf[r0, sl]
                for k in range(1, K):
                    m = jnp.maximum(m, buf[r0 + k, sl])
                obuf[i, sl] = m
            return 0

        lax.fori_loop(0, CHUNK, node_body, 0)
        pltpu.sync_copy(obuf, out_hbm.at[pl.ds(base + j * CHUNK, CHUNK)])

    issue(0, gbuf_a, sem_a)
    issue(1, gbuf_b, sem_b)

    def chunk_body(jj, _):
        j0 = jj * 2
        wait(gbuf_a, sem_a)
        compute_store(j0, gbuf_a, obuf_a)
        issue(jnp.minimum(j0 + 2, NCHUNK - 1), gbuf_a, sem_a)
        wait(gbuf_b, sem_b)
        compute_store(j0 + 1, gbuf_b, obuf_b)
        issue(jnp.minimum(j0 + 3, NCHUNK - 1), gbuf_b, sem_b)
        return 0

    lax.fori_loop(0, NCHUNK // 2, chunk_body, 0)
    wait(gbuf_a, sem_a)
    wait(gbuf_b, sem_b)


def kernel(x, idx, W, b):
    x2 = x[0]
    idx_flat = idx[0].astype(jnp.int32).reshape(-1)
    idx3 = jnp.pad(idx_flat, (0, (NPAD - N) * K)).reshape(NW, NCHUNK, IDX_PER_CHUNK)
    b2 = b.reshape(1, D)

    yt = pl.pallas_call(
        _mm_body,
        grid=(NBLK,),
        in_specs=[
            pl.BlockSpec((D, NB), lambda i: (0, i)),
            pl.BlockSpec((D, 2 * D), lambda i: (0, 0)),
        ],
        out_specs=pl.BlockSpec((NB, D), lambda i: (i, 0)),
        out_shape=jax.ShapeDtypeStruct((NPAD, D), jnp.float32),
    )(x2, W)

    mesh = plsc.VectorSubcoreMesh(core_axis_name="c", subcore_axis_name="s")
    mt = pl.kernel(
        _sc_knn,
        out_type=jax.ShapeDtypeStruct((NPAD, D), jnp.float32),
        mesh=mesh,
        scratch_types=[
            pltpu.VMEM((NCHUNK, IDX_PER_CHUNK), jnp.int32),
            pltpu.VMEM((IDX_PER_CHUNK, D), jnp.float32),
            pltpu.VMEM((IDX_PER_CHUNK, D), jnp.float32),
            pltpu.VMEM((CHUNK, D), jnp.float32),
            pltpu.VMEM((CHUNK, D), jnp.float32),
            pltpu.VMEM_SHARED((NPAD, D), jnp.float32),
            pltpu.SemaphoreType.DMA,
            pltpu.SemaphoreType.DMA,
        ],
    )(yt, idx3)

    out = pl.pallas_call(
        _add_body,
        grid=(NBLK,),
        in_specs=[
            pl.BlockSpec((D, NB), lambda i: (0, i)),
            pl.BlockSpec((NB, D), lambda i: (i, 0)),
            pl.BlockSpec((D, 2 * D), lambda i: (0, 0)),
            pl.BlockSpec((1, D), lambda i: (0, 0)),
        ],
        out_specs=pl.BlockSpec((D, NB), lambda i: (0, i)),
        out_shape=jax.ShapeDtypeStruct((D, N), jnp.float32),
    )(x2, mt, W, b2)

    return out[None]


# speedup vs baseline: 3.1104x; 1.1069x over previous
import functools

import jax
import jax.numpy as jnp
from jax import lax
from jax.experimental import pallas as pl
from jax.experimental.pallas import tpu as pltpu
from jax.experimental.pallas import tpu_sc as plsc

B = 1
D = 128
N = 10000
K = 16

NW = 32
NPAD = 10240
NODES_PER_W = NPAD // NW
CHUNK = 8
NCHUNK = NODES_PER_W // CHUNK
IDX_PER_CHUNK = CHUNK * K
NB = 1024
NBLK = NPAD // NB
ROWS_PER_TILE = NPAD // 16


def _mm_body(x_ref, w_ref, yt_ref):
    xb = x_ref[...]
    w1 = w_ref[:, :D]
    dn = (((0,), (1,)), ((), ()))
    yt_ref[...] = lax.dot_general(xb, w1, dn, preferred_element_type=jnp.float32)


def _add_body(x_ref, m_ref, w_ref, b_ref, o_ref):
    xb = x_ref[...]
    wz = w_ref[:, D:] - w_ref[:, :D]
    z = lax.dot_general(wz, xb, (((1,), (0,)), ((), ())),
                        preferred_element_type=jnp.float32) + b_ref[...].T
    o_ref[...] = xb + jnp.maximum(z + m_ref[...].T, 0.0)


def _sc_knn(yt_hbm, idx_hbm, out_hbm, idx_v, gbuf_a, gbuf_b, obuf_a, obuf_b,
            yt_sp, sem_a, sem_b):
    cid = lax.axis_index("c")
    sid = lax.axis_index("s")
    wid = sid * 2 + cid
    base = wid * NODES_PER_W
    pltpu.sync_copy(yt_hbm.at[pl.ds(sid * ROWS_PER_TILE, ROWS_PER_TILE)],
                    yt_sp.at[pl.ds(sid * ROWS_PER_TILE, ROWS_PER_TILE)])
    pltpu.sync_copy(idx_hbm.at[wid], idx_v)
    plsc.subcore_barrier()

    def issue(jc, buf, sem):
        pltpu.async_copy(yt_sp.at[idx_v.at[jc]], buf, sem)

    def wait(buf, sem):
        pltpu.make_async_copy(yt_sp.at[pl.ds(0, IDX_PER_CHUNK)], buf, sem).wait()

    def compute_store(j, buf, obuf):
        def node_body(i, _):
            r0 = i * K
            for g in range(D // 16):
                sl = pl.ds(g * 16, 16)
                m = buf[r0, sl]
                for k in range(1, K):
                    m = jnp.maximum(m, buf[r0 + k, sl])
                obuf[i, sl] = m
            return 0

        lax.fori_loop(0, CHUNK, node_body, 0)
        pltpu.sync_copy(obuf, out_hbm.at[pl.ds(base + j * CHUNK, CHUNK)])

    issue(0, gbuf_a, sem_a)
    issue(1, gbuf_b, sem_b)

    def chunk_body(jj, _):
        j0 = jj * 2
        wait(gbuf_a, sem_a)
        compute_store(j0, gbuf_a, obuf_a)
        issue(jnp.minimum(j0 + 2, NCHUNK - 1), gbuf_a, sem_a)
        wait(gbuf_b, sem_b)
        compute_store(j0 + 1, gbuf_b, obuf_b)
        issue(jnp.minimum(j0 + 3, NCHUNK - 1), gbuf_b, sem_b)
        return 0

    lax.fori_loop(0, NCHUNK // 2, chunk_body, 0)
    wait(gbuf_a, sem_a)
    wait(gbuf_b, sem_b)


def kernel(x, idx, W, b):
    x2 = x[0]
    idx_flat = idx[0].astype(jnp.int32).reshape(-1)
    idx3 = jnp.pad(idx_flat, (0, (NPAD - N) * K)).reshape(NW, NCHUNK, IDX_PER_CHUNK)
    b2 = b.reshape(1, D)

    yt = pl.pallas_call(
        _mm_body,
        grid=(NBLK,),
        in_specs=[
            pl.BlockSpec((D, NB), lambda i: (0, i)),
            pl.BlockSpec((D, 2 * D), lambda i: (0, 0)),
        ],
        out_specs=pl.BlockSpec((NB, D), lambda i: (i, 0)),
        out_shape=jax.ShapeDtypeStruct((NPAD, D), jnp.float32),
    )(x2, W)

    mesh = plsc.VectorSubcoreMesh(core_axis_name="c", subcore_axis_name="s")
    mt = pl.kernel(
        _sc_knn,
        out_type=jax.ShapeDtypeStruct((NPAD, D), jnp.float32),
        mesh=mesh,
        scratch_types=[
            pltpu.VMEM((NCHUNK, IDX_PER_CHUNK), jnp.int32),
            pltpu.VMEM((IDX_PER_CHUNK, D), jnp.float32),
            pltpu.VMEM((IDX_PER_CHUNK, D), jnp.float32),
            pltpu.VMEM((CHUNK, D), jnp.float32),
            pltpu.VMEM((CHUNK, D), jnp.float32),
            pltpu.VMEM_SHARED((NPAD, D), jnp.float32),
            pltpu.SemaphoreType.DMA,
            pltpu.SemaphoreType.DMA,
        ],
    )(yt, idx3)

    out = pl.pallas_call(
        _add_body,
        grid=(NBLK,),
        in_specs=[
            pl.BlockSpec((D, NB), lambda i: (0, i)),
            pl.BlockSpec((NB, D), lambda i: (i, 0)),
            pl.BlockSpec((D, 2 * D), lambda i: (0, 0)),
            pl.BlockSpec((1, D), lambda i: (0, 0)),
        ],
        out_specs=pl.BlockSpec((D, NB), lambda i: (0, i)),
        out_shape=jax.ShapeDtypeStruct((D, N), jnp.float32),
    )(x2, mt, W, b2)

    return out[None]


# speedup vs baseline: 3.2516x; 1.0454x over previous
import functools

import jax
import jax.numpy as jnp
from jax import lax
from jax.experimental import pallas as pl
from jax.experimental.pallas import tpu as pltpu
from jax.experimental.pallas import tpu_sc as plsc

B = 1
D = 128
N = 10000
K = 16

NW = 32
NPAD = 10240
NODES_PER_W = NPAD // NW
CHUNK = 8
NCHUNK = NODES_PER_W // CHUNK
IDX_PER_CHUNK = CHUNK * K
NB = 2048
NBLK = NPAD // NB
ROWS_PER_TILE = NPAD // 16


def _mm_body(x_ref, w_ref, yt_ref):
    xb = x_ref[...]
    w1 = w_ref[:, :D]
    dn = (((0,), (1,)), ((), ()))
    yt_ref[...] = lax.dot_general(xb, w1, dn, preferred_element_type=jnp.float32)


def _add_body(x_ref, m_ref, w_ref, b_ref, o_ref):
    xb = x_ref[...]
    wz = w_ref[:, D:] - w_ref[:, :D]
    z = lax.dot_general(wz, xb, (((1,), (0,)), ((), ())),
                        preferred_element_type=jnp.float32) + b_ref[...].T
    o_ref[...] = xb + jnp.maximum(z + m_ref[...].T, 0.0)


def _sc_knn(yt_hbm, idx_hbm, out_hbm, idx_v, gbuf_a, gbuf_b, obuf_a, obuf_b,
            yt_sp, sem_a, sem_b):
    cid = lax.axis_index("c")
    sid = lax.axis_index("s")
    wid = sid * 2 + cid
    base = wid * NODES_PER_W
    pltpu.sync_copy(yt_hbm.at[pl.ds(sid * ROWS_PER_TILE, ROWS_PER_TILE)],
                    yt_sp.at[pl.ds(sid * ROWS_PER_TILE, ROWS_PER_TILE)])
    pltpu.sync_copy(idx_hbm.at[wid], idx_v)
    plsc.subcore_barrier()

    def issue(jc, buf, sem):
        pltpu.async_copy(yt_sp.at[idx_v.at[jc]], buf, sem)

    def wait(buf, sem):
        pltpu.make_async_copy(yt_sp.at[pl.ds(0, IDX_PER_CHUNK)], buf, sem).wait()

    def compute_store(j, buf, obuf):
        def node_body(i, _):
            r0 = i * K
            for g in range(D // 16):
                sl = pl.ds(g * 16, 16)
                m = buf[r0, sl]
                for k in range(1, K):
                    m = jnp.maximum(m, buf[r0 + k, sl])
                obuf[i, sl] = m
            return 0

        lax.fori_loop(0, CHUNK, node_body, 0)
        pltpu.sync_copy(obuf, out_hbm.at[pl.ds(base + j * CHUNK, CHUNK)])

    issue(0, gbuf_a, sem_a)
    issue(1, gbuf_b, sem_b)

    def chunk_body(jj, _):
        j0 = jj * 2
        wait(gbuf_a, sem_a)
        compute_store(j0, gbuf_a, obuf_a)
        issue(jnp.minimum(j0 + 2, NCHUNK - 1), gbuf_a, sem_a)
        wait(gbuf_b, sem_b)
        compute_store(j0 + 1, gbuf_b, obuf_b)
        issue(jnp.minimum(j0 + 3, NCHUNK - 1), gbuf_b, sem_b)
        return 0

    lax.fori_loop(0, NCHUNK // 2, chunk_body, 0)
    wait(gbuf_a, sem_a)
    wait(gbuf_b, sem_b)


def kernel(x, idx, W, b):
    x2 = x[0]
    idx_flat = idx[0].astype(jnp.int32).reshape(-1)
    idx3 = jnp.pad(idx_flat, (0, (NPAD - N) * K)).reshape(NW, NCHUNK, IDX_PER_CHUNK)
    b2 = b.reshape(1, D)

    yt = pl.pallas_call(
        _mm_body,
        grid=(NBLK,),
        in_specs=[
            pl.BlockSpec((D, NB), lambda i: (0, i)),
            pl.BlockSpec((D, 2 * D), lambda i: (0, 0)),
        ],
        out_specs=pl.BlockSpec((NB, D), lambda i: (i, 0)),
        out_shape=jax.ShapeDtypeStruct((NPAD, D), jnp.float32),
    )(x2, W)

    mesh = plsc.VectorSubcoreMesh(core_axis_name="c", subcore_axis_name="s")
    mt = pl.kernel(
        _sc_knn,
        out_type=jax.ShapeDtypeStruct((NPAD, D), jnp.float32),
        mesh=mesh,
        scratch_types=[
            pltpu.VMEM((NCHUNK, IDX_PER_CHUNK), jnp.int32),
            pltpu.VMEM((IDX_PER_CHUNK, D), jnp.float32),
            pltpu.VMEM((IDX_PER_CHUNK, D), jnp.float32),
            pltpu.VMEM((CHUNK, D), jnp.float32),
            pltpu.VMEM((CHUNK, D), jnp.float32),
            pltpu.VMEM_SHARED((NPAD, D), jnp.float32),
            pltpu.SemaphoreType.DMA,
            pltpu.SemaphoreType.DMA,
        ],
    )(yt, idx3)

    out = pl.pallas_call(
        _add_body,
        grid=(NBLK,),
        in_specs=[
            pl.BlockSpec((D, NB), lambda i: (0, i)),
            pl.BlockSpec((NB, D), lambda i: (i, 0)),
            pl.BlockSpec((D, 2 * D), lambda i: (0, 0)),
            pl.BlockSpec((1, D), lambda i: (0, 0)),
        ],
        out_specs=pl.BlockSpec((D, NB), lambda i: (0, i)),
        out_shape=jax.ShapeDtypeStruct((D, N), jnp.float32),
    )(x2, mt, W, b2)

    return out[None]


# speedup vs baseline: 3.4276x; 1.0541x over previous
import functools

import jax
import jax.numpy as jnp
from jax import lax
from jax.experimental import pallas as pl
from jax.experimental.pallas import tpu as pltpu
from jax.experimental.pallas import tpu_sc as plsc

B = 1
D = 128
N = 10000
K = 16

NW = 32
NPAD = 10240
NODES_PER_W = NPAD // NW
CHUNK = 8
NCHUNK = NODES_PER_W // CHUNK
IDX_PER_CHUNK = CHUNK * K
NB = 5120
NBLK = NPAD // NB
ROWS_PER_TILE = NPAD // 16


def _mm_body(x_ref, w_ref, yt_ref):
    xb = x_ref[...]
    w1 = w_ref[:, :D]
    dn = (((0,), (1,)), ((), ()))
    yt_ref[...] = lax.dot_general(xb, w1, dn, preferred_element_type=jnp.float32)


def _add_body(x_ref, m_ref, w_ref, b_ref, o_ref):
    xb = x_ref[...]
    wz = w_ref[:, D:] - w_ref[:, :D]
    z = lax.dot_general(wz, xb, (((1,), (0,)), ((), ())),
                        preferred_element_type=jnp.float32) + b_ref[...].T
    o_ref[...] = xb + jnp.maximum(z + m_ref[...].T, 0.0)


def _sc_knn(yt_hbm, idx_hbm, out_hbm, idx_v, gbuf_a, gbuf_b, obuf_a, obuf_b,
            yt_sp, sem_a, sem_b):
    cid = lax.axis_index("c")
    sid = lax.axis_index("s")
    wid = sid * 2 + cid
    base = wid * NODES_PER_W
    pltpu.sync_copy(yt_hbm.at[pl.ds(sid * ROWS_PER_TILE, ROWS_PER_TILE)],
                    yt_sp.at[pl.ds(sid * ROWS_PER_TILE, ROWS_PER_TILE)])
    pltpu.sync_copy(idx_hbm.at[wid], idx_v)
    plsc.subcore_barrier()

    def issue(jc, buf, sem):
        pltpu.async_copy(yt_sp.at[idx_v.at[jc]], buf, sem)

    def wait(buf, sem):
        pltpu.make_async_copy(yt_sp.at[pl.ds(0, IDX_PER_CHUNK)], buf, sem).wait()

    def compute_store(j, buf, obuf):
        def node_body(i, _):
            r0 = i * K
            for g in range(D // 16):
                sl = pl.ds(g * 16, 16)
                m = buf[r0, sl]
                for k in range(1, K):
                    m = jnp.maximum(m, buf[r0 + k, sl])
                obuf[i, sl] = m
            return 0

        lax.fori_loop(0, CHUNK, node_body, 0)
        pltpu.sync_copy(obuf, out_hbm.at[pl.ds(base + j * CHUNK, CHUNK)])

    issue(0, gbuf_a, sem_a)
    issue(1, gbuf_b, sem_b)

    def chunk_body(jj, _):
        j0 = jj * 2
        wait(gbuf_a, sem_a)
        compute_store(j0, gbuf_a, obuf_a)
        issue(jnp.minimum(j0 + 2, NCHUNK - 1), gbuf_a, sem_a)
        wait(gbuf_b, sem_b)
        compute_store(j0 + 1, gbuf_b, obuf_b)
        issue(jnp.minimum(j0 + 3, NCHUNK - 1), gbuf_b, sem_b)
        return 0

    lax.fori_loop(0, NCHUNK // 2, chunk_body, 0)
    wait(gbuf_a, sem_a)
    wait(gbuf_b, sem_b)


def kernel(x, idx, W, b):
    x2 = x[0]
    idx_flat = idx[0].astype(jnp.int32).reshape(-1)
    idx3 = jnp.pad(idx_flat, (0, (NPAD - N) * K)).reshape(NW, NCHUNK, IDX_PER_CHUNK)
    b2 = b.reshape(1, D)

    yt = pl.pallas_call(
        _mm_body,
        grid=(NBLK,),
        in_specs=[
            pl.BlockSpec((D, NB), lambda i: (0, i)),
            pl.BlockSpec((D, 2 * D), lambda i: (0, 0)),
        ],
        out_specs=pl.BlockSpec((NB, D), lambda i: (i, 0)),
        out_shape=jax.ShapeDtypeStruct((NPAD, D), jnp.float32),
    )(x2, W)

    mesh = plsc.VectorSubcoreMesh(core_axis_name="c", subcore_axis_name="s")
    mt = pl.kernel(
        _sc_knn,
        out_type=jax.ShapeDtypeStruct((NPAD, D), jnp.float32),
        mesh=mesh,
        scratch_types=[
            pltpu.VMEM((NCHUNK, IDX_PER_CHUNK), jnp.int32),
            pltpu.VMEM((IDX_PER_CHUNK, D), jnp.float32),
            pltpu.VMEM((IDX_PER_CHUNK, D), jnp.float32),
            pltpu.VMEM((CHUNK, D), jnp.float32),
            pltpu.VMEM((CHUNK, D), jnp.float32),
            pltpu.VMEM_SHARED((NPAD, D), jnp.float32),
            pltpu.SemaphoreType.DMA,
            pltpu.SemaphoreType.DMA,
        ],
    )(yt, idx3)

    out = pl.pallas_call(
        _add_body,
        grid=(NBLK,),
        in_specs=[
            pl.BlockSpec((D, NB), lambda i: (0, i)),
            pl.BlockSpec((NB, D), lambda i: (i, 0)),
            pl.BlockSpec((D, 2 * D), lambda i: (0, 0)),
            pl.BlockSpec((1, D), lambda i: (0, 0)),
        ],
        out_specs=pl.BlockSpec((D, NB), lambda i: (0, i)),
        out_shape=jax.ShapeDtypeStruct((D, N), jnp.float32),
    )(x2, mt, W, b2)

    return out[None]
